# baseline (speedup 1.0000x reference)
import functools
import math

import jax
import jax.numpy as jnp
from jax import lax
from jax.experimental import pallas as pl
from jax.experimental.pallas import tpu as pltpu
from jax.experimental.pallas import tpu_sc as plsc

D_MODEL = 64
SCALE = math.sqrt(D_MODEL)

_NC = 2
_NS = 16
_NW = _NC * _NS
_CHUNK = 128
_LANES = 16
_BLK = 80


@functools.lru_cache(maxsize=None)
def _make_format_kernel(V):
    _CPB = 384
    _PRB = _CPB // 2
    nblk = V // _CPB
    tail_cols = V - nblk * _CPB

    mesh = plsc.VectorSubcoreMesh(core_axis_name="c", subcore_axis_name="s")

    @functools.partial(
        pl.kernel,
        mesh=mesh,
        compiler_params=pltpu.CompilerParams(needs_layout_passes=False),
        out_type=jax.ShapeDtypeStruct((V // 2, 2 * D_MODEL), jnp.float32),
        scratch_types=[
            pltpu.VMEM((2, D_MODEL, _CPB), jnp.float32),
            pltpu.VMEM((2, _PRB, 2 * D_MODEL), jnp.float32),
            pltpu.VMEM((D_MODEL, tail_cols if tail_cols else 1), jnp.float32),
            pltpu.SemaphoreType.DMA,
            pltpu.SemaphoreType.DMA,
            pltpu.SemaphoreType.DMA,
            pltpu.SemaphoreType.DMA,
        ],
    )
    def k1(tT_hbm, tail_hbm, scr_hbm, strip_v, out_v, tail_v, gi0, gi1, go0, go1):
        gi = (gi0, gi1)
        go = (go0, go1)
        wid = lax.axis_index("s") * _NC + lax.axis_index("c")
        my_nblk = (nblk - wid + _NW - 1) // _NW

        def col0(t):
            return pl.multiple_of((wid + _NW * t) * _CPB, _CPB)

        def row0(t):
            return pl.multiple_of((wid + _NW * t) * _PRB, _PRB)

        def start_in(t, b):
            c0 = col0(t)
            for d in range(D_MODEL):
                pltpu.async_copy(
                    tT_hbm.at[d, pl.ds(c0, _CPB)], strip_v.at[b, d], gi[b]
                )

        def wait_in(b):
            pltpu.make_async_copy(
                tT_hbm.at[:, pl.ds(0, _CPB)], strip_v.at[b], gi[b]
            ).wait()

        def start_out(t, b):
            pltpu.async_copy(
                out_v.at[b], scr_hbm.at[pl.ds(row0(t), _PRB)], go[b]
            )

        def wait_out(b):
            pltpu.make_async_copy(
                out_v.at[b], scr_hbm.at[pl.ds(0, _PRB)], go[b]
            ).wait()

        start_in(0, 0)

        row_idx = [
            jnp.arange(d0 * _LANES, (d0 + 1) * _LANES, dtype=jnp.int32)
            for d0 in range(D_MODEL // _LANES)
        ]

        def transpose_block(b, npair, strip):
            def quad_body(i, c):
                cb = jnp.full((_LANES,), 8 * i, dtype=jnp.int32)
                for u in range(4):
                    p = 4 * i + u
                    for h in range(2):
                        col = cb + (2 * u + h)
                        for d0 in range(D_MODEL // _LANES):
                            vals = plsc.load_gather(strip, [row_idx[d0], col])
                            out_v[b, p, pl.ds(h * D_MODEL + d0 * _LANES, _LANES)] = vals
                return c

            lax.fori_loop(0, npair // 4, quad_body, 0)

        def transpose_tail(npair):
            def pair_body(p, c):
                for h in range(2):
                    col = jnp.full((_LANES,), 2 * p + h, dtype=jnp.int32)
                    for d0 in range(D_MODEL // _LANES):
                        vals = plsc.load_gather(tail_v, [row_idx[d0], col])
                        out_v[0, p, pl.ds(h * D_MODEL + d0 * _LANES, _LANES)] = vals
                return c

            lax.fori_loop(0, npair, pair_body, 0)

        def do_block(t, b):
            @pl.when(t + 1 < my_nblk)
            def _():
                start_in(t + 1, 1 - b)

            wait_in(b)

            @pl.when(t >= 2)
            def _():
                wait_out(b)

            transpose_block(b, _PRB, strip_v.at[b])
            start_out(t, b)

        def outer(tt, carry):
            for b in range(2):
                t = 2 * tt + b

                @pl.when(t < my_nblk)
                def _():
                    do_block(t, b)

            return carry

        lax.fori_loop(0, (my_nblk + 1) // 2, outer, 0)
        for b in range(2):
            wait_out(b)

        if tail_cols:
            @pl.when(wid == _NW - 1)
            def _():
                pltpu.sync_copy(tail_hbm, tail_v)
                transpose_tail(tail_cols // 2)
                pltpu.sync_copy(
                    out_v.at[0, pl.ds(0, tail_cols // 2)],
                    scr_hbm.at[pl.ds(nblk * _PRB, tail_cols // 2)],
                )

    return k1


@functools.lru_cache(maxsize=None)
def _make_lookup_kernel(B, V):
    assert B % (_NW * _CHUNK) == 0
    rows_per_w = B // _NW
    nch = rows_per_w // _CHUNK

    mesh = plsc.VectorSubcoreMesh(core_axis_name="c", subcore_axis_name="s")

    @functools.partial(
        pl.kernel,
        mesh=mesh,
        out_type=jax.ShapeDtypeStruct((B, D_MODEL), jnp.float32),
        scratch_types=[
            pltpu.VMEM((nch, _CHUNK), jnp.int32),
            pltpu.VMEM((2, _CHUNK), jnp.int32),
            pltpu.VMEM((2, _CHUNK, 2 * D_MODEL), jnp.float32),
            pltpu.VMEM((2, _CHUNK, D_MODEL), jnp.float32),
            pltpu.SemaphoreType.DMA,
            pltpu.SemaphoreType.DMA,
            pltpu.SemaphoreType.DMA,
            pltpu.SemaphoreType.DMA,
        ],
    )
    def k2(x_hbm, t2_hbm, out_hbm, idx_v, pair_v, in_v, out_v, g0, g1, o0, o1):
        gsems = (g0, g1)
        osems = (o0, o1)
        wid = lax.axis_index("s") * _NC + lax.axis_index("c")
        base_idx_row = wid * nch
        base_out = wid * rows_per_w
        pltpu.sync_copy(x_hbm.at[pl.ds(base_idx_row, nch)], idx_v)

        def start_gather(j, b):
            def pair_body(kk, c):
                sl = pl.ds(kk * _LANES, _LANES)
                pair_v[b, sl] = lax.shift_right_logical(idx_v[j, sl], 1)
                return c

            lax.fori_loop(0, _CHUNK // _LANES, pair_body, 0)
            pltpu.async_copy(t2_hbm.at[pair_v.at[b]], in_v.at[b], gsems[b])

        start_gather(0, 0)

        def process_chunk(j, b):
            @pl.when(j + 1 < nch)
            def _():
                start_gather(j + 1, 1 - b)

            pltpu.make_async_copy(
                t2_hbm.at[pair_v.at[b]], in_v.at[b], gsems[b]
            ).wait()

            @pl.when(j >= 2)
            def _():
                pltpu.make_async_copy(
                    out_v.at[b], out_hbm.at[pl.ds(base_out, _CHUNK)], osems[b]
                ).wait()

            def group_body(g, c):
                idxv = idx_v[j, pl.ds(g * _LANES, _LANES)]
                base = (idxv & 1) * D_MODEL
                for ll in range(_LANES):
                    bb = base[ll]
                    r = g * _LANES + ll
                    for kk in range(D_MODEL // _LANES):
                        o = kk * _LANES
                        out_v[b, r, pl.ds(o, _LANES)] = (
                            in_v[b, r, pl.ds(bb + o, _LANES)] * SCALE
                        )
                return c

            lax.fori_loop(0, _CHUNK // _LANES, group_body, 0)
            pltpu.async_copy(
                out_v.at[b],
                out_hbm.at[pl.ds(base_out + j * _CHUNK, _CHUNK)],
                osems[b],
            )

        def outer_body(jj, carry):
            for b in range(2):
                process_chunk(2 * jj + b, b)
            return carry

        lax.fori_loop(0, nch // 2, outer_body, 0)
        for b in range(2):
            pltpu.make_async_copy(
                out_v.at[b], out_hbm.at[pl.ds(base_out, _CHUNK)], osems[b]
            ).wait()

    return k2


def kernel(x, table):
    B = x.size
    V = table.shape[0]
    x2 = x.reshape(-1, _CHUNK).astype(jnp.int32)
    tT = table.T
    tail_cols = V % 256
    tail = tT[:, V - tail_cols:] if tail_cols else tT[:, :1]
    t2 = _make_format_kernel(V)(tT, tail)
    out = _make_lookup_kernel(B, V)(x2, t2)
    return out.reshape(x.shape + (D_MODEL,))


# speedup vs baseline: 1.0217x; 1.0217x over previous
import functools
import math

import jax
import jax.numpy as jnp
from jax import lax
from jax.experimental import pallas as pl
from jax.experimental.pallas import tpu as pltpu
from jax.experimental.pallas import tpu_sc as plsc

D_MODEL = 64
SCALE = math.sqrt(D_MODEL)

_NC = 2
_NS = 16
_NW = _NC * _NS
_CHUNK = 128
_LANES = 16
_BLK = 80


@functools.lru_cache(maxsize=None)
def _make_format_kernel(V):
    _CPB = 384
    _PRB = _CPB // 2
    _NTC = _CPB // 128
    _NT = _NTC * (D_MODEL // 8)
    nblk = V // _CPB
    tail_cols = V - nblk * _CPB

    mesh = plsc.VectorSubcoreMesh(core_axis_name="c", subcore_axis_name="s")

    @functools.partial(
        pl.kernel,
        mesh=mesh,
        compiler_params=pltpu.CompilerParams(needs_layout_passes=False),
        out_type=jax.ShapeDtypeStruct((V // 2, 2 * D_MODEL), jnp.float32),
        scratch_types=[
            pltpu.VMEM((2, _NT, 8, 128), jnp.float32),
            pltpu.VMEM((2, _PRB, 2 * D_MODEL), jnp.float32),
            pltpu.VMEM((D_MODEL, tail_cols if tail_cols else 1), jnp.float32),
            pltpu.SemaphoreType.DMA,
            pltpu.SemaphoreType.DMA,
            pltpu.SemaphoreType.DMA,
            pltpu.SemaphoreType.DMA,
        ],
    )
    def k1(tT_hbm, tail_hbm, scr_hbm, strip_v, out_v, tail_v, gi0, gi1, go0, go1):
        gi = (gi0, gi1)
        go = (go0, go1)
        wid = lax.axis_index("s") * _NC + lax.axis_index("c")
        my_nblk = (nblk - wid + _NW - 1) // _NW

        def col0(t):
            return pl.multiple_of((wid + _NW * t) * _CPB, _CPB)

        def row0(t):
            return pl.multiple_of((wid + _NW * t) * _PRB, _PRB)

        def start_in(t, b):
            c0 = col0(t)
            for q in range(D_MODEL // 8):
                for jj in range(_NTC):
                    pltpu.async_copy(
                        tT_hbm.at[pl.ds(8 * q, 8), pl.ds(c0 + 128 * jj, 128)],
                        strip_v.at[b, q * _NTC + jj],
                        gi[b],
                    )

        def wait_in(b):
            for i in range(_NT):
                pltpu.make_async_copy(
                    tT_hbm.at[pl.ds(0, 8), pl.ds(0, 128)],
                    strip_v.at[b, i],
                    gi[b],
                ).wait()

        def start_out(t, b):
            pltpu.async_copy(
                out_v.at[b], scr_hbm.at[pl.ds(row0(t), _PRB)], go[b]
            )

        def wait_out(b):
            pltpu.make_async_copy(
                out_v.at[b], scr_hbm.at[pl.ds(0, _PRB)], go[b]
            ).wait()

        start_in(0, 0)

        row_idx = [
            jnp.arange(d0 * _LANES, (d0 + 1) * _LANES, dtype=jnp.int32)
            for d0 in range(D_MODEL // _LANES)
        ]
        lanes = jnp.arange(_LANES, dtype=jnp.int32)
        tile_idx = [((row_idx[k] >> 3) * _NTC) for k in range(D_MODEL // _LANES)]
        sub_idx = lanes & 7

        def transpose_block(b, npair, strip):
            def quad_body(i, c):
                for u in range(4):
                    p = 4 * i + u
                    for h in range(2):
                        col = 2 * p + h
                        cj = jnp.full((_LANES,), col >> 7, dtype=jnp.int32)
                        ck = jnp.full((_LANES,), col & 127, dtype=jnp.int32)
                        for k in range(D_MODEL // _LANES):
                            vals = plsc.load_gather(
                                strip, [tile_idx[k] + cj, sub_idx, ck]
                            )
                            out_v[b, p, pl.ds(h * D_MODEL + k * _LANES, _LANES)] = vals
                return c

            lax.fori_loop(0, npair // 4, quad_body, 0)

        def transpose_tail(npair):
            def pair_body(p, c):
                for h in range(2):
                    col = jnp.full((_LANES,), 2 * p + h, dtype=jnp.int32)
                    for d0 in range(D_MODEL // _LANES):
                        vals = plsc.load_gather(tail_v, [row_idx[d0], col])
                        out_v[0, p, pl.ds(h * D_MODEL + d0 * _LANES, _LANES)] = vals
                return c

            lax.fori_loop(0, npair, pair_body, 0)

        def do_block(t, b):
            @pl.when(t + 1 < my_nblk)
            def _():
                start_in(t + 1, 1 - b)

            wait_in(b)

            @pl.when(t >= 2)
            def _():
                wait_out(b)

            transpose_block(b, _PRB, strip_v.at[b])
            start_out(t, b)

        def outer(tt, carry):
            for b in range(2):
                t = 2 * tt + b

                @pl.when(t < my_nblk)
                def _():
                    do_block(t, b)

            return carry

        lax.fori_loop(0, (my_nblk + 1) // 2, outer, 0)
        for b in range(2):
            wait_out(b)

        if tail_cols:
            @pl.when(wid == _NW - 1)
            def _():
                pltpu.sync_copy(tail_hbm, tail_v)
                transpose_tail(tail_cols // 2)
                pltpu.sync_copy(
                    out_v.at[0, pl.ds(0, tail_cols // 2)],
                    scr_hbm.at[pl.ds(nblk * _PRB, tail_cols // 2)],
                )

    return k1


@functools.lru_cache(maxsize=None)
def _make_lookup_kernel(B, V):
    assert B % (_NW * _CHUNK) == 0
    rows_per_w = B // _NW
    nch = rows_per_w // _CHUNK

    mesh = plsc.VectorSubcoreMesh(core_axis_name="c", subcore_axis_name="s")

    @functools.partial(
        pl.kernel,
        mesh=mesh,
        out_type=jax.ShapeDtypeStruct((B, D_MODEL), jnp.float32),
        scratch_types=[
            pltpu.VMEM((nch, _CHUNK), jnp.int32),
            pltpu.VMEM((2, _CHUNK), jnp.int32),
            pltpu.VMEM((2, _CHUNK, 2 * D_MODEL), jnp.float32),
            pltpu.VMEM((2, _CHUNK, D_MODEL), jnp.float32),
            pltpu.SemaphoreType.DMA,
            pltpu.SemaphoreType.DMA,
            pltpu.SemaphoreType.DMA,
            pltpu.SemaphoreType.DMA,
        ],
    )
    def k2(x_hbm, t2_hbm, out_hbm, idx_v, pair_v, in_v, out_v, g0, g1, o0, o1):
        gsems = (g0, g1)
        osems = (o0, o1)
        wid = lax.axis_index("s") * _NC + lax.axis_index("c")
        base_idx_row = wid * nch
        base_out = wid * rows_per_w
        pltpu.sync_copy(x_hbm.at[pl.ds(base_idx_row, nch)], idx_v)

        def start_gather(j, b):
            def pair_body(kk, c):
                sl = pl.ds(kk * _LANES, _LANES)
                pair_v[b, sl] = lax.shift_right_logical(idx_v[j, sl], 1)
                return c

            lax.fori_loop(0, _CHUNK // _LANES, pair_body, 0)
            pltpu.async_copy(t2_hbm.at[pair_v.at[b]], in_v.at[b], gsems[b])

        start_gather(0, 0)

        def process_chunk(j, b):
            @pl.when(j + 1 < nch)
            def _():
                start_gather(j + 1, 1 - b)

            pltpu.make_async_copy(
                t2_hbm.at[pair_v.at[b]], in_v.at[b], gsems[b]
            ).wait()

            @pl.when(j >= 2)
            def _():
                pltpu.make_async_copy(
                    out_v.at[b], out_hbm.at[pl.ds(base_out, _CHUNK)], osems[b]
                ).wait()

            def group_body(g, c):
                idxv = idx_v[j, pl.ds(g * _LANES, _LANES)]
                base = (idxv & 1) * D_MODEL
                for ll in range(_LANES):
                    bb = base[ll]
                    r = g * _LANES + ll
                    for kk in range(D_MODEL // _LANES):
                        o = kk * _LANES
                        out_v[b, r, pl.ds(o, _LANES)] = (
                            in_v[b, r, pl.ds(bb + o, _LANES)] * SCALE
                        )
                return c

            lax.fori_loop(0, _CHUNK // _LANES, group_body, 0)
            pltpu.async_copy(
                out_v.at[b],
                out_hbm.at[pl.ds(base_out + j * _CHUNK, _CHUNK)],
                osems[b],
            )

        def outer_body(jj, carry):
            for b in range(2):
                process_chunk(2 * jj + b, b)
            return carry

        lax.fori_loop(0, nch // 2, outer_body, 0)
        for b in range(2):
            pltpu.make_async_copy(
                out_v.at[b], out_hbm.at[pl.ds(base_out, _CHUNK)], osems[b]
            ).wait()

    return k2


def kernel(x, table):
    B = x.size
    V = table.shape[0]
    x2 = x.reshape(-1, _CHUNK).astype(jnp.int32)
    tT = table.T
    tail_cols = V % 256
    tail = tT[:, V - tail_cols:] if tail_cols else tT[:, :1]
    t2 = _make_format_kernel(V)(tT, tail)
    out = _make_lookup_kernel(B, V)(x2, t2)
    return out.reshape(x.shape + (D_MODEL,))


# speedup vs baseline: 1.7544x; 1.7171x over previous
import functools
import math

import jax
import jax.numpy as jnp
from jax import lax
from jax.experimental import pallas as pl
from jax.experimental.pallas import tpu as pltpu
from jax.experimental.pallas import tpu_sc as plsc

D_MODEL = 64
SCALE = math.sqrt(D_MODEL)

_NC = 2
_NS = 16
_NW = _NC * _NS
_CHUNK = 128
_LANES = 16
_BLK = 80


@functools.lru_cache(maxsize=None)
def _make_format_kernel(V):
    _CPB = 384
    _PRB = _CPB // 2
    _NTC = _CPB // 128
    _NT = _NTC * (D_MODEL // 8)
    nblk = V // _CPB
    tail_cols = V - nblk * _CPB

    mesh = plsc.VectorSubcoreMesh(core_axis_name="c", subcore_axis_name="s")

    @functools.partial(
        pl.kernel,
        mesh=mesh,
        compiler_params=pltpu.CompilerParams(needs_layout_passes=False),
        out_type=jax.ShapeDtypeStruct((V // 2, 2 * D_MODEL), jnp.float32),
        scratch_types=[
            pltpu.VMEM((2, _NT, 8, 128), jnp.float32),
            pltpu.VMEM((2, _PRB, 2 * D_MODEL), jnp.float32),
            pltpu.VMEM((D_MODEL, tail_cols if tail_cols else 1), jnp.float32),
            pltpu.SemaphoreType.DMA,
            pltpu.SemaphoreType.DMA,
            pltpu.SemaphoreType.DMA,
            pltpu.SemaphoreType.DMA,
        ],
    )
    def k1(tT_hbm, tail_hbm, scr_hbm, strip_v, out_v, tail_v, gi0, gi1, go0, go1):
        gi = (gi0, gi1)
        go = (go0, go1)
        wid = lax.axis_index("s") * _NC + lax.axis_index("c")
        my_nblk = (nblk - wid + _NW - 1) // _NW

        def col0(t):
            return pl.multiple_of((wid + _NW * t) * _CPB, _CPB)

        def row0(t):
            return pl.multiple_of((wid + _NW * t) * _PRB, _PRB)

        def start_in(t, b):
            c0 = col0(t)
            for q in range(D_MODEL // 8):
                for jj in range(_NTC):
                    pltpu.async_copy(
                        tT_hbm.at[pl.ds(8 * q, 8), pl.ds(c0 + 128 * jj, 128)],
                        strip_v.at[b, q * _NTC + jj],
                        gi[b],
                    )

        def wait_in(b):
            for i in range(_NT):
                pltpu.make_async_copy(
                    tT_hbm.at[pl.ds(0, 8), pl.ds(0, 128)],
                    strip_v.at[b, i],
                    gi[b],
                ).wait()

        def start_out(t, b):
            pltpu.async_copy(
                out_v.at[b], scr_hbm.at[pl.ds(row0(t), _PRB)], go[b]
            )

        def wait_out(b):
            pltpu.make_async_copy(
                out_v.at[b], scr_hbm.at[pl.ds(0, _PRB)], go[b]
            ).wait()

        start_in(0, 0)

        row_idx = [
            jnp.arange(d0 * _LANES, (d0 + 1) * _LANES, dtype=jnp.int32)
            for d0 in range(D_MODEL // _LANES)
        ]
        lanes = jnp.arange(_LANES, dtype=jnp.int32)
        tile_idx = [((row_idx[k] >> 3) * _NTC) for k in range(D_MODEL // _LANES)]
        sub_idx = lanes & 7

        def transpose_block(b, npair, strip):
            def quad_body(i, c):
                for u in range(4):
                    p = 4 * i + u
                    for h in range(2):
                        col = 2 * p + h
                        cj = jnp.full((_LANES,), col >> 7, dtype=jnp.int32)
                        ck = jnp.full((_LANES,), col & 127, dtype=jnp.int32)
                        for k in range(D_MODEL // _LANES):
                            vals = plsc.load_gather(
                                strip, [tile_idx[k] + cj, sub_idx, ck]
                            )
                            out_v[b, p, pl.ds(h * D_MODEL + k * _LANES, _LANES)] = vals
                return c

            lax.fori_loop(0, npair // 4, quad_body, 0)

        def transpose_tail(npair):
            def pair_body(p, c):
                for h in range(2):
                    col = jnp.full((_LANES,), 2 * p + h, dtype=jnp.int32)
                    for d0 in range(D_MODEL // _LANES):
                        vals = plsc.load_gather(tail_v, [row_idx[d0], col])
                        out_v[0, p, pl.ds(h * D_MODEL + d0 * _LANES, _LANES)] = vals
                return c

            lax.fori_loop(0, npair, pair_body, 0)

        def do_block(t, b):
            @pl.when(t + 1 < my_nblk)
            def _():
                start_in(t + 1, 1 - b)

            wait_in(b)

            @pl.when(t >= 2)
            def _():
                wait_out(b)

            transpose_block(b, _PRB, strip_v.at[b])
            start_out(t, b)

        def outer(tt, carry):
            for b in range(2):
                t = 2 * tt + b

                @pl.when(t < my_nblk)
                def _():
                    do_block(t, b)

            return carry

        lax.fori_loop(0, (my_nblk + 1) // 2, outer, 0)
        for b in range(2):
            wait_out(b)

        if tail_cols:
            @pl.when(wid == _NW - 1)
            def _():
                pltpu.sync_copy(tail_hbm, tail_v)
                transpose_tail(tail_cols // 2)
                pltpu.sync_copy(
                    out_v.at[0, pl.ds(0, tail_cols // 2)],
                    scr_hbm.at[pl.ds(nblk * _PRB, tail_cols // 2)],
                )

    return k1


@functools.lru_cache(maxsize=None)
def _make_lookup_kernel(B, V):
    assert B % (_NW * _CHUNK) == 0
    rows_per_w = B // _NW
    nch = rows_per_w // _CHUNK

    mesh = plsc.VectorSubcoreMesh(core_axis_name="c", subcore_axis_name="s")

    @functools.partial(
        pl.kernel,
        mesh=mesh,
        out_type=jax.ShapeDtypeStruct((B, D_MODEL), jnp.float32),
        scratch_types=[
            pltpu.VMEM((nch, _CHUNK), jnp.int32),
            pltpu.VMEM((2, _CHUNK), jnp.int32),
            pltpu.VMEM((2, _CHUNK, 2 * D_MODEL), jnp.float32),
            pltpu.VMEM((2, _CHUNK, D_MODEL), jnp.float32),
            pltpu.SemaphoreType.DMA,
            pltpu.SemaphoreType.DMA,
            pltpu.SemaphoreType.DMA,
            pltpu.SemaphoreType.DMA,
        ],
    )
    def k2(x_hbm, t2_hbm, out_hbm, idx_v, pair_v, in_v, out_v, g0, g1, o0, o1):
        gsems = (g0, g1)
        osems = (o0, o1)
        wid = lax.axis_index("s") * _NC + lax.axis_index("c")
        base_idx_row = wid * nch
        base_out = wid * rows_per_w
        pltpu.sync_copy(x_hbm.at[pl.ds(base_idx_row, nch)], idx_v)

        def start_gather(j, b):
            def pair_body(kk, c):
                sl = pl.ds(kk * _LANES, _LANES)
                pair_v[b, sl] = lax.shift_right_logical(idx_v[j, sl], 1)
                return c

            lax.fori_loop(0, _CHUNK // _LANES, pair_body, 0)
            pltpu.async_copy(t2_hbm.at[pair_v.at[b]], in_v.at[b], gsems[b])

        start_gather(0, 0)

        def process_chunk(j, b):
            @pl.when(j + 1 < nch)
            def _():
                start_gather(j + 1, 1 - b)

            pltpu.make_async_copy(
                t2_hbm.at[pair_v.at[b]], in_v.at[b], gsems[b]
            ).wait()

            @pl.when(j >= 2)
            def _():
                pltpu.make_async_copy(
                    out_v.at[b], out_hbm.at[pl.ds(base_out, _CHUNK)], osems[b]
                ).wait()

            def group_body(g, c):
                idxv = idx_v[j, pl.ds(g * _LANES, _LANES)]
                base = (idxv & 1) * D_MODEL
                for ll in range(_LANES):
                    bb = base[ll]
                    r = g * _LANES + ll
                    for kk in range(D_MODEL // _LANES):
                        o = kk * _LANES
                        out_v[b, r, pl.ds(o, _LANES)] = (
                            in_v[b, r, pl.ds(bb + o, _LANES)] * SCALE
                        )
                return c

            lax.fori_loop(0, _CHUNK // _LANES, group_body, 0)
            pltpu.async_copy(
                out_v.at[b],
                out_hbm.at[pl.ds(base_out + j * _CHUNK, _CHUNK)],
                osems[b],
            )

        def outer_body(jj, carry):
            for b in range(2):
                process_chunk(2 * jj + b, b)
            return carry

        lax.fori_loop(0, nch // 2, outer_body, 0)
        for b in range(2):
            pltpu.make_async_copy(
                out_v.at[b], out_hbm.at[pl.ds(base_out, _CHUNK)], osems[b]
            ).wait()

    return k2


def kernel(x, table):
    B = x.size
    V = table.shape[0]
    x2 = x.reshape(-1, _CHUNK).astype(jnp.int32)
    t2 = table.reshape(V // 2, 2 * D_MODEL)
    out = _make_lookup_kernel(B, V)(x2, t2)
    return out.reshape(x.shape + (D_MODEL,))
